# speedup vs baseline: 62.0063x; 62.0063x over previous
import jax
import jax.numpy as jnp
from jax.experimental import pallas as pl

B, P, G, C = 16, 43008, 32, 2
THRESH = 0.35
NEGPOS = 7
V0, V1 = 0.1, 0.2


def _sl1(d):
    ad = jnp.abs(d)
    return jnp.where(ad < 1.0, 0.5 * ad * ad, ad - 0.5)


def _phase_a_body(x_ref, pri_ref, tt_ref, part_ref, rank_ref):
    X = x_ref[0]
    PR = pri_ref[...]
    TT = tt_ref[0]

    px, py = PR[0:1], PR[1:2]
    pw, ph = PR[2:3], PR[3:4]
    bx1, by1 = px - pw * 0.5, py - ph * 0.5
    bx2, by2 = px + pw * 0.5, py + ph * 0.5
    area_b = (bx2 - bx1) * (by2 - by1)

    tx1 = TT[0:1].reshape(G, 1)
    ty1 = TT[1:2].reshape(G, 1)
    tx2 = TT[2:3].reshape(G, 1)
    ty2 = TT[3:4].reshape(G, 1)
    area_a = (tx2 - tx1) * (ty2 - ty1)

    ix = jnp.clip(jnp.minimum(tx2, bx2) - jnp.maximum(tx1, bx1), 0.0, None)
    iy = jnp.clip(jnp.minimum(ty2, by2) - jnp.maximum(ty1, by1), 0.0, None)
    inter = ix * iy
    ov = inter / (area_a + area_b - inter)

    g_iota = jax.lax.broadcasted_iota(jnp.int32, (G, 1), 0)
    p_iota = jax.lax.broadcasted_iota(jnp.int32, (1, P), 1)
    BIG = jnp.int32(2**30)

    bto = jnp.max(ov, axis=0, keepdims=True)
    bti = jnp.min(jnp.where(ov == bto, g_iota, BIG), axis=0, keepdims=True)

    bpo = jnp.max(ov, axis=1, keepdims=True)
    bpi = jnp.min(jnp.where(ov == bpo, p_iota, BIG), axis=1, keepdims=True)

    forced = jnp.max(jnp.where(bpi == p_iota, g_iota, -1), axis=0, keepdims=True)
    is_f = forced >= 0
    bto = jnp.where(is_f, 2.0, bto)
    bti = jnp.where(is_f, forced, bti)

    onehot = (g_iota == bti).astype(jnp.float32)
    M = jax.lax.dot_general(
        TT, onehot, (((1,), (0,)), ((), ())),
        precision=jax.lax.Precision.HIGHEST,
        preferred_element_type=jnp.float32)

    conf = jnp.where(bto < THRESH, 0.0, M[14:15])
    pos = conf != 0.0
    posl = conf > 0.0
    posf = pos.astype(jnp.float32)
    poslf = posl.astype(jnp.float32)

    mx1, my1, mx2, my2 = M[0:1], M[1:2], M[2:3], M[3:4]
    g_cx = ((mx1 + mx2) * 0.5 - px) / (V0 * pw)
    g_cy = ((my1 + my2) * 0.5 - py) / (V0 * ph)
    g_w = jnp.log((mx2 - mx1) / pw) / V1
    g_h = jnp.log((my2 - my1) / ph) / V1
    ll = _sl1(X[0:1] - g_cx) + _sl1(X[1:2] - g_cy) \
        + _sl1(X[2:3] - g_w) + _sl1(X[3:4] - g_h)
    loss_l = jnp.sum(ll * posf)

    lm = jnp.zeros((1, P), jnp.float32)
    for j in range(5):
        gx = (M[4 + 2 * j:5 + 2 * j] - px) / (V0 * pw)
        gy = (M[5 + 2 * j:6 + 2 * j] - py) / (V0 * ph)
        lm = lm + _sl1(X[4 + 2 * j:5 + 2 * j] - gx) \
            + _sl1(X[5 + 2 * j:6 + 2 * j] - gy)
    loss_lm = jnp.sum(lm * poslf)

    a, b = X[14:15], X[15:16]
    m = jnp.maximum(a, b)
    lse = m + jnp.log(jnp.exp(a - m) + jnp.exp(b - m))
    ce = lse - jnp.where(pos, b, a)
    ce_pos = jnp.sum(ce * posf)
    rank = jnp.where(pos, 0.0, ce)

    n_pos = jnp.sum(posf)
    n_posl = jnp.sum(poslf)

    rows = jnp.concatenate([
        jnp.broadcast_to(loss_l, (1, 128)),
        jnp.broadcast_to(loss_lm, (1, 128)),
        jnp.broadcast_to(n_pos, (1, 128)),
        jnp.broadcast_to(n_posl, (1, 128)),
        jnp.broadcast_to(ce_pos, (1, 128)),
        jnp.zeros((3, 128), jnp.float32),
    ], axis=0)
    part_ref[0] = rows
    rank_ref[0] = rank


def _phase_b_body(rank_ref, k_ref, out_ref):
    rows = jnp.concatenate([rank_ref[i] for i in range(B)], axis=0)
    bits = jax.lax.bitcast_convert_type(rows, jnp.int32)
    k = k_ref[:, 0:1]

    lo = jnp.zeros((B, 1), jnp.int32)
    hi = jnp.full((B, 1), jnp.int32(0x7F800000))
    for _ in range(31):
        mid = lo + (hi - lo + 1) // 2
        cnt = jnp.sum((bits >= mid).astype(jnp.int32), axis=1, keepdims=True)
        take = cnt >= k
        lo = jnp.where(take, mid, lo)
        hi = jnp.where(take, hi, mid - 1)

    tval = jax.lax.bitcast_convert_type(lo, jnp.float32)
    gt = bits > lo
    cnt_gt = jnp.sum(gt.astype(jnp.int32), axis=1, keepdims=True)
    sum_gt = jnp.sum(jnp.where(gt, rows, 0.0), axis=1, keepdims=True)
    topk = sum_gt + (k - cnt_gt).astype(jnp.float32) * tval
    topk = jnp.where(k > 0, topk, 0.0)
    out_ref[...] = jnp.broadcast_to(topk, (B, 128))


def kernel(loc_data, conf_data, landm_data, priors, targets):
    x = jnp.concatenate([
        loc_data.transpose(0, 2, 1),
        landm_data.transpose(0, 2, 1),
        conf_data.transpose(0, 2, 1),
    ], axis=1)
    pri = jnp.concatenate(
        [priors.T, jnp.zeros((4, P), jnp.float32)], axis=0)
    tt = jnp.concatenate(
        [targets.transpose(0, 2, 1),
         jnp.zeros((B, 1, G), jnp.float32)], axis=1)

    part, rank = pl.pallas_call(
        _phase_a_body,
        grid=(B,),
        in_specs=[
            pl.BlockSpec((1, 16, P), lambda b: (b, 0, 0)),
            pl.BlockSpec((8, P), lambda b: (0, 0)),
            pl.BlockSpec((1, 16, G), lambda b: (b, 0, 0)),
        ],
        out_specs=[
            pl.BlockSpec((1, 8, 128), lambda b: (b, 0, 0)),
            pl.BlockSpec((1, 1, P), lambda b: (b, 0, 0)),
        ],
        out_shape=[
            jax.ShapeDtypeStruct((B, 8, 128), jnp.float32),
            jax.ShapeDtypeStruct((B, 1, P), jnp.float32),
        ],
    )(x, pri, tt)

    s = part[:, :, 0]
    num_pos = s[:, 2].astype(jnp.int32)
    k = jnp.minimum(NEGPOS * num_pos, P - 1)
    k_arr = jnp.broadcast_to(k[:, None], (B, 128)).astype(jnp.int32)

    topk = pl.pallas_call(
        _phase_b_body,
        in_specs=[
            pl.BlockSpec((B, 1, P), lambda: (0, 0, 0)),
            pl.BlockSpec((B, 128), lambda: (0, 0)),
        ],
        out_specs=pl.BlockSpec((B, 128), lambda: (0, 0)),
        out_shape=jax.ShapeDtypeStruct((B, 128), jnp.float32),
    )(rank, k_arr)

    loss_c = jnp.sum(s[:, 4]) + jnp.sum(topk[:, 0])
    N = jnp.maximum(jnp.sum(s[:, 2]), 1.0)
    N1 = jnp.maximum(jnp.sum(s[:, 3]), 1.0)
    return (jnp.sum(s[:, 0]) / N, loss_c / N, jnp.sum(s[:, 1]) / N1)


# speedup vs baseline: 68.4143x; 1.1033x over previous
import jax
import jax.numpy as jnp
from jax.experimental import pallas as pl
from jax.experimental.pallas import tpu as pltpu

B, P, G, C = 16, 43008, 32, 2
THRESH = 0.35
NEGPOS = 7
V0, V1 = 0.1, 0.2


def _sl1(d):
    ad = jnp.abs(d)
    return jnp.where(ad < 1.0, 0.5 * ad * ad, ad - 0.5)


def _body(x_ref, pri_ref, tt_ref, out_ref, rank_s, stat_s):
    b = pl.program_id(0)
    X = x_ref[0]
    PRI = pri_ref[...]
    TT = tt_ref[0]

    tx1 = TT[0:1].reshape(G, 1)
    ty1 = TT[1:2].reshape(G, 1)
    tx2 = TT[2:3].reshape(G, 1)
    ty2 = TT[3:4].reshape(G, 1)
    area_a = (tx2 - tx1) * (ty2 - ty1)

    ix = jnp.clip(jnp.minimum(tx2, PRI[6:7]) - jnp.maximum(tx1, PRI[4:5]),
                  0.0, None)
    iy = jnp.clip(jnp.minimum(ty2, PRI[7:8]) - jnp.maximum(ty1, PRI[5:6]),
                  0.0, None)
    inter = ix * iy
    ov = inter / (area_a + PRI[8:9] - inter)

    g_iota = jax.lax.broadcasted_iota(jnp.int32, (G, 1), 0)
    p_iota = jax.lax.broadcasted_iota(jnp.int32, (1, P), 1)
    BIG = jnp.int32(2**30)

    bto = jnp.max(ov, axis=0, keepdims=True)
    bti = jnp.min(jnp.where(ov == bto, g_iota, BIG), axis=0, keepdims=True)

    bpo = jnp.max(ov, axis=1, keepdims=True)
    bpi = jnp.min(jnp.where(ov == bpo, p_iota, BIG), axis=1, keepdims=True)

    forced = jnp.max(jnp.where(bpi == p_iota, g_iota, -1), axis=0,
                     keepdims=True)
    is_f = forced >= 0
    bto = jnp.where(is_f, 2.0, bto)
    bti = jnp.where(is_f, forced, bti)

    onehot = (g_iota == bti).astype(jnp.float32)
    M = jax.lax.dot_general(
        TT, onehot, (((1,), (0,)), ((), ())),
        precision=jax.lax.Precision.HIGHEST,
        preferred_element_type=jnp.float32)

    conf = jnp.where(bto < THRESH, 0.0, M[14:15])
    pos = conf != 0.0
    posf = pos.astype(jnp.float32)
    poslf = (conf > 0.0).astype(jnp.float32)

    pxy = PRI[0:2]
    iv0 = PRI[9:11]
    ipwh = PRI[11:13]
    g_cxy = ((M[0:2] + M[2:4]) * 0.5 - pxy) * iv0
    g_wh = jnp.log((M[2:4] - M[0:2]) * ipwh) * (1.0 / V1)
    dl = X[0:4] - jnp.concatenate([g_cxy, g_wh], axis=0)
    ll_v = jnp.sum(_sl1(dl), axis=0, keepdims=True)
    loss_l = jnp.sum(ll_v * posf)

    pxy10 = jnp.concatenate([pxy] * 5, axis=0)
    iv10 = jnp.concatenate([iv0] * 5, axis=0)
    g_lm = (M[4:14] - pxy10) * iv10
    lm_v = jnp.sum(_sl1(X[4:14] - g_lm), axis=0, keepdims=True)
    loss_lm = jnp.sum(lm_v * poslf)

    a, c = X[14:15], X[15:16]
    m = jnp.maximum(a, c)
    lse = m + jnp.log(jnp.exp(a - m) + jnp.exp(c - m))
    ce = lse - jnp.where(pos, c, a)
    ce_pos = jnp.sum(ce * posf)
    rank = jnp.where(pos, 0.0, ce)

    n_pos = jnp.sum(posf)
    n_posl = jnp.sum(poslf)

    lane = jax.lax.broadcasted_iota(jnp.int32, (1, 128), 1)
    srow = jnp.where(lane == 0, loss_l,
                     jnp.where(lane == 1, loss_lm,
                               jnp.where(lane == 2, n_pos,
                                         jnp.where(lane == 3, n_posl,
                                                   ce_pos))))
    row16 = jax.lax.broadcasted_iota(jnp.int32, (B, 1), 0)
    stat_s[...] = jnp.where(row16 == b, jnp.broadcast_to(srow, (B, 128)),
                            stat_s[...])
    rank_s[...] = jnp.where(row16 == b, jnp.broadcast_to(rank, (B, P)),
                            rank_s[...])

    @pl.when(b == B - 1)
    def _finalize():
        rows = rank_s[...]
        bits = jax.lax.bitcast_convert_type(rows, jnp.int32)
        npos_col = stat_s[:, 2:3]
        k = jnp.minimum(NEGPOS * npos_col.astype(jnp.int32), P - 1)

        lo = jnp.zeros((B, 1), jnp.int32)
        hi = jnp.full((B, 1), jnp.int32(0x7F800000))
        for _ in range(31):
            mid = lo + (hi - lo + 1) // 2
            cnt = jnp.sum((bits >= mid).astype(jnp.int32), axis=1,
                          keepdims=True)
            take = cnt >= k
            lo = jnp.where(take, mid, lo)
            hi = jnp.where(take, hi, mid - 1)

        tval = jax.lax.bitcast_convert_type(lo, jnp.float32)
        gt = bits > lo
        cnt_gt = jnp.sum(gt.astype(jnp.int32), axis=1, keepdims=True)
        sum_gt = jnp.sum(jnp.where(gt, rows, 0.0), axis=1, keepdims=True)
        topk = sum_gt + (k - cnt_gt).astype(jnp.float32) * tval
        topk = jnp.where(k > 0, topk, 0.0)
        topk_total = jnp.sum(topk)

        S = jnp.sum(stat_s[...], axis=0, keepdims=True)
        lane1 = jax.lax.broadcasted_iota(jnp.int32, (1, 128), 1)

        def pick(i):
            return jnp.sum(jnp.where(lane1 == i, S, 0.0))

        N = jnp.maximum(pick(2), 1.0)
        N1 = jnp.maximum(pick(3), 1.0)
        out_ref[...] = jnp.concatenate([
            jnp.broadcast_to(pick(0) / N, (1, 128)),
            jnp.broadcast_to((pick(4) + topk_total) / N, (1, 128)),
            jnp.broadcast_to(pick(1) / N1, (1, 128)),
            jnp.zeros((5, 128), jnp.float32),
        ], axis=0)


def kernel(loc_data, conf_data, landm_data, priors, targets):
    x = jnp.concatenate([
        loc_data.transpose(0, 2, 1),
        landm_data.transpose(0, 2, 1),
        conf_data.transpose(0, 2, 1),
    ], axis=1)

    pt = priors.T
    pxy, pwh = pt[0:2], pt[2:4]
    pf1 = pxy - pwh * 0.5
    pf2 = pxy + pwh * 0.5
    area_b = (pf2[0:1] - pf1[0:1]) * (pf2[1:2] - pf1[1:2])
    pri = jnp.concatenate([
        pxy,
        pwh,
        pf1, pf2,
        area_b,
        1.0 / (V0 * pwh),
        1.0 / pwh,
        jnp.zeros((3, P), jnp.float32),
    ], axis=0)

    tt = jnp.concatenate(
        [targets.transpose(0, 2, 1),
         jnp.zeros((B, 1, G), jnp.float32)], axis=1)

    out = pl.pallas_call(
        _body,
        grid=(B,),
        in_specs=[
            pl.BlockSpec((1, 16, P), lambda b: (b, 0, 0)),
            pl.BlockSpec((16, P), lambda b: (0, 0)),
            pl.BlockSpec((1, 16, G), lambda b: (b, 0, 0)),
        ],
        out_specs=pl.BlockSpec((8, 128), lambda b: (0, 0)),
        out_shape=jax.ShapeDtypeStruct((8, 128), jnp.float32),
        scratch_shapes=[
            pltpu.VMEM((B, P), jnp.float32),
            pltpu.VMEM((B, 128), jnp.float32),
        ],
    )(x, pri, tt)

    return (out[0, 0], out[1, 0], out[2, 0])


# speedup vs baseline: 69.3984x; 1.0144x over previous
import jax
import jax.numpy as jnp
from jax.experimental import pallas as pl
from jax.experimental.pallas import tpu as pltpu

B, P, G, C = 16, 43008, 32, 2
THRESH = 0.35
NEGPOS = 7
V0, V1 = 0.1, 0.2


def _sl1(d):
    ad = jnp.abs(d)
    return jnp.where(ad < 1.0, 0.5 * ad * ad, ad - 0.5)


def _match_body(pri_ref, tt_ref, bti_ref, bto_ref):
    PRI = pri_ref[...]
    TT = tt_ref[0]

    tx1 = TT[0:1].reshape(G, 1)
    ty1 = TT[1:2].reshape(G, 1)
    tx2 = TT[2:3].reshape(G, 1)
    ty2 = TT[3:4].reshape(G, 1)
    area_a = (tx2 - tx1) * (ty2 - ty1)

    ix = jnp.clip(jnp.minimum(tx2, PRI[6:7]) - jnp.maximum(tx1, PRI[4:5]),
                  0.0, None)
    iy = jnp.clip(jnp.minimum(ty2, PRI[7:8]) - jnp.maximum(ty1, PRI[5:6]),
                  0.0, None)
    inter = ix * iy
    ov = inter / (area_a + PRI[8:9] - inter)

    g_iota = jax.lax.broadcasted_iota(jnp.int32, (G, 1), 0)
    p_iota = jax.lax.broadcasted_iota(jnp.int32, (1, P), 1)
    BIG = jnp.int32(2**30)

    bto = jnp.max(ov, axis=0, keepdims=True)
    bti = jnp.min(jnp.where(ov == bto, g_iota, BIG), axis=0, keepdims=True)

    bpo = jnp.max(ov, axis=1, keepdims=True)
    bpi = jnp.min(jnp.where(ov == bpo, p_iota, BIG), axis=1, keepdims=True)

    forced = jnp.max(jnp.where(bpi == p_iota, g_iota, -1), axis=0,
                     keepdims=True)
    is_f = forced >= 0
    bto_ref[0] = jnp.where(is_f, 2.0, bto)
    bti_ref[0] = jnp.where(is_f, forced, bti)


def _loss_body(x_ref, pri_ref, tt_ref, bti_ref, bto_ref, out_ref,
               rank_s, stat_s):
    b = pl.program_id(0)
    X = x_ref[0]
    PRI = pri_ref[...]
    TT = tt_ref[0]
    bti = bti_ref[0]
    bto = bto_ref[0]

    g_iota = jax.lax.broadcasted_iota(jnp.int32, (G, 1), 0)

    onehot = (g_iota == bti).astype(jnp.float32)
    M = jax.lax.dot_general(
        TT, onehot, (((1,), (0,)), ((), ())),
        precision=jax.lax.Precision.HIGHEST,
        preferred_element_type=jnp.float32)

    conf = jnp.where(bto < THRESH, 0.0, M[14:15])
    pos = conf != 0.0
    posf = pos.astype(jnp.float32)
    poslf = (conf > 0.0).astype(jnp.float32)

    pxy = PRI[0:2]
    iv0 = PRI[9:11]
    ipwh = PRI[11:13]
    g_cxy = ((M[0:2] + M[2:4]) * 0.5 - pxy) * iv0
    g_wh = jnp.log((M[2:4] - M[0:2]) * ipwh) * (1.0 / V1)
    dl = X[0:4] - jnp.concatenate([g_cxy, g_wh], axis=0)
    ll_v = jnp.sum(_sl1(dl), axis=0, keepdims=True)
    loss_l = jnp.sum(ll_v * posf)

    pxy10 = jnp.concatenate([pxy] * 5, axis=0)
    iv10 = jnp.concatenate([iv0] * 5, axis=0)
    g_lm = (M[4:14] - pxy10) * iv10
    lm_v = jnp.sum(_sl1(X[4:14] - g_lm), axis=0, keepdims=True)
    loss_lm = jnp.sum(lm_v * poslf)

    a, c = X[14:15], X[15:16]
    m = jnp.maximum(a, c)
    lse = m + jnp.log(jnp.exp(a - m) + jnp.exp(c - m))
    ce = lse - jnp.where(pos, c, a)
    ce_pos = jnp.sum(ce * posf)
    rank = jnp.where(pos, 0.0, ce)

    n_pos = jnp.sum(posf)
    n_posl = jnp.sum(poslf)

    lane = jax.lax.broadcasted_iota(jnp.int32, (1, 128), 1)
    srow = jnp.where(lane == 0, loss_l,
                     jnp.where(lane == 1, loss_lm,
                               jnp.where(lane == 2, n_pos,
                                         jnp.where(lane == 3, n_posl,
                                                   ce_pos))))
    row16 = jax.lax.broadcasted_iota(jnp.int32, (B, 1), 0)
    stat_s[...] = jnp.where(row16 == b, jnp.broadcast_to(srow, (B, 128)),
                            stat_s[...])
    rank_s[...] = jnp.where(row16 == b, jnp.broadcast_to(rank, (B, P)),
                            rank_s[...])

    @pl.when(b == B - 1)
    def _finalize():
        rows = rank_s[...]
        bits = jax.lax.bitcast_convert_type(rows, jnp.int32)
        npos_col = stat_s[:, 2:3]
        k = jnp.minimum(NEGPOS * npos_col.astype(jnp.int32), P - 1)

        lo = jnp.zeros((B, 1), jnp.int32)
        hi = jnp.full((B, 1), jnp.int32(0x7F800000))
        for _ in range(31):
            mid = lo + (hi - lo + 1) // 2
            cnt = jnp.sum((bits >= mid).astype(jnp.int32), axis=1,
                          keepdims=True)
            take = cnt >= k
            lo = jnp.where(take, mid, lo)
            hi = jnp.where(take, hi, mid - 1)

        tval = jax.lax.bitcast_convert_type(lo, jnp.float32)
        gt = bits > lo
        cnt_gt = jnp.sum(gt.astype(jnp.int32), axis=1, keepdims=True)
        sum_gt = jnp.sum(jnp.where(gt, rows, 0.0), axis=1, keepdims=True)
        topk = sum_gt + (k - cnt_gt).astype(jnp.float32) * tval
        topk = jnp.where(k > 0, topk, 0.0)
        topk_total = jnp.sum(topk)

        S = jnp.sum(stat_s[...], axis=0, keepdims=True)
        lane1 = jax.lax.broadcasted_iota(jnp.int32, (1, 128), 1)

        def pick(i):
            return jnp.sum(jnp.where(lane1 == i, S, 0.0))

        N = jnp.maximum(pick(2), 1.0)
        N1 = jnp.maximum(pick(3), 1.0)
        out_ref[...] = jnp.concatenate([
            jnp.broadcast_to(pick(0) / N, (1, 128)),
            jnp.broadcast_to((pick(4) + topk_total) / N, (1, 128)),
            jnp.broadcast_to(pick(1) / N1, (1, 128)),
            jnp.zeros((5, 128), jnp.float32),
        ], axis=0)


def kernel(loc_data, conf_data, landm_data, priors, targets):
    x = jnp.concatenate([
        loc_data.transpose(0, 2, 1),
        landm_data.transpose(0, 2, 1),
        conf_data.transpose(0, 2, 1),
    ], axis=1)

    pt = priors.T
    pxy, pwh = pt[0:2], pt[2:4]
    pf1 = pxy - pwh * 0.5
    pf2 = pxy + pwh * 0.5
    area_b = (pf2[0:1] - pf1[0:1]) * (pf2[1:2] - pf1[1:2])
    pri = jnp.concatenate([
        pxy,
        pwh,
        pf1, pf2,
        area_b,
        1.0 / (V0 * pwh),
        1.0 / pwh,
        jnp.zeros((3, P), jnp.float32),
    ], axis=0)

    tt = jnp.concatenate(
        [targets.transpose(0, 2, 1),
         jnp.zeros((B, 1, G), jnp.float32)], axis=1)

    bti, bto = pl.pallas_call(
        _match_body,
        grid=(B,),
        in_specs=[
            pl.BlockSpec((16, P), lambda b: (0, 0)),
            pl.BlockSpec((1, 16, G), lambda b: (b, 0, 0)),
        ],
        out_specs=[
            pl.BlockSpec((1, 1, P), lambda b: (b, 0, 0)),
            pl.BlockSpec((1, 1, P), lambda b: (b, 0, 0)),
        ],
        out_shape=[
            jax.ShapeDtypeStruct((B, 1, P), jnp.int32),
            jax.ShapeDtypeStruct((B, 1, P), jnp.float32),
        ],
    )(pri, tt)

    out = pl.pallas_call(
        _loss_body,
        grid=(B,),
        in_specs=[
            pl.BlockSpec((1, 16, P), lambda b: (b, 0, 0)),
            pl.BlockSpec((16, P), lambda b: (0, 0)),
            pl.BlockSpec((1, 16, G), lambda b: (b, 0, 0)),
            pl.BlockSpec((1, 1, P), lambda b: (b, 0, 0)),
            pl.BlockSpec((1, 1, P), lambda b: (b, 0, 0)),
        ],
        out_specs=pl.BlockSpec((8, 128), lambda b: (0, 0)),
        out_shape=jax.ShapeDtypeStruct((8, 128), jnp.float32),
        scratch_shapes=[
            pltpu.VMEM((B, P), jnp.float32),
            pltpu.VMEM((B, 128), jnp.float32),
        ],
    )(x, pri, tt, bti, bto)

    return (out[0, 0], out[1, 0], out[2, 0])


# speedup vs baseline: 82.2977x; 1.1859x over previous
import jax
import jax.numpy as jnp
from jax.experimental import pallas as pl
from jax.experimental.pallas import tpu as pltpu

B, P, G, C = 16, 43008, 32, 2
THRESH = 0.35
NEGPOS = 7
V0, V1 = 0.1, 0.2


def _sl1(d):
    ad = jnp.abs(d)
    return jnp.where(ad < 1.0, 0.5 * ad * ad, ad - 0.5)


def _match_body(pri_ref, tt_ref, bti_ref, bto_ref):
    PRI = pri_ref[...]
    TT = tt_ref[0]

    tx1 = TT[0:1].reshape(G, 1)
    ty1 = TT[1:2].reshape(G, 1)
    tx2 = TT[2:3].reshape(G, 1)
    ty2 = TT[3:4].reshape(G, 1)
    area_a = (tx2 - tx1) * (ty2 - ty1)

    ix = jnp.clip(jnp.minimum(tx2, PRI[6:7]) - jnp.maximum(tx1, PRI[4:5]),
                  0.0, None)
    iy = jnp.clip(jnp.minimum(ty2, PRI[7:8]) - jnp.maximum(ty1, PRI[5:6]),
                  0.0, None)
    inter = ix * iy
    ov = inter / (area_a + PRI[8:9] - inter)

    g_iota = jax.lax.broadcasted_iota(jnp.int32, (G, 1), 0)
    p_iota = jax.lax.broadcasted_iota(jnp.int32, (1, P), 1)
    BIG = jnp.int32(2**30)

    bto = jnp.max(ov, axis=0, keepdims=True)
    bti = jnp.min(jnp.where(ov == bto, g_iota, BIG), axis=0, keepdims=True)

    bpo = jnp.max(ov, axis=1, keepdims=True)
    bpi = jnp.min(jnp.where(ov == bpo, p_iota, BIG), axis=1, keepdims=True)

    forced = jnp.max(jnp.where(bpi == p_iota, g_iota, -1), axis=0,
                     keepdims=True)
    is_f = forced >= 0
    bto_ref[0] = jnp.where(is_f, 2.0, bto)
    bti_ref[0] = jnp.where(is_f, forced, bti)


def _loss_body(xl_ref, xm_ref, xc_ref, pri_ref, tt_ref, bti_ref, bto_ref,
               out_ref, rank_s, stat_s):
    b = pl.program_id(0)
    XL = xl_ref[0]
    XM = xm_ref[0]
    XC = xc_ref[0]
    PRI = pri_ref[...]
    TT = tt_ref[0]
    bti = bti_ref[0]
    bto = bto_ref[0]

    g_iota = jax.lax.broadcasted_iota(jnp.int32, (G, 1), 0)

    onehot = (g_iota == bti).astype(jnp.float32)
    M = jax.lax.dot_general(
        TT, onehot, (((1,), (0,)), ((), ())),
        precision=jax.lax.Precision.HIGHEST,
        preferred_element_type=jnp.float32)

    conf = jnp.where(bto < THRESH, 0.0, M[14:15])
    pos = conf != 0.0
    posf = pos.astype(jnp.float32)
    poslf = (conf > 0.0).astype(jnp.float32)

    pxy = PRI[0:2]
    iv0 = PRI[9:11]
    ipwh = PRI[11:13]
    g_cxy = ((M[0:2] + M[2:4]) * 0.5 - pxy) * iv0
    g_wh = jnp.log((M[2:4] - M[0:2]) * ipwh) * (1.0 / V1)
    dl = XL - jnp.concatenate([g_cxy, g_wh], axis=0)
    ll_v = jnp.sum(_sl1(dl), axis=0, keepdims=True)
    loss_l = jnp.sum(ll_v * posf)

    pxy10 = jnp.concatenate([pxy] * 5, axis=0)
    iv10 = jnp.concatenate([iv0] * 5, axis=0)
    g_lm = (M[4:14] - pxy10) * iv10
    lm_v = jnp.sum(_sl1(XM - g_lm), axis=0, keepdims=True)
    loss_lm = jnp.sum(lm_v * poslf)

    a, c = XC[0:1], XC[1:2]
    m = jnp.maximum(a, c)
    lse = m + jnp.log(jnp.exp(a - m) + jnp.exp(c - m))
    ce = lse - jnp.where(pos, c, a)
    ce_pos = jnp.sum(ce * posf)
    rank = jnp.where(pos, 0.0, ce)

    n_pos = jnp.sum(posf)
    n_posl = jnp.sum(poslf)

    lane = jax.lax.broadcasted_iota(jnp.int32, (1, 128), 1)
    srow = jnp.where(lane == 0, loss_l,
                     jnp.where(lane == 1, loss_lm,
                               jnp.where(lane == 2, n_pos,
                                         jnp.where(lane == 3, n_posl,
                                                   ce_pos))))
    row16 = jax.lax.broadcasted_iota(jnp.int32, (B, 1), 0)
    stat_s[...] = jnp.where(row16 == b, jnp.broadcast_to(srow, (B, 128)),
                            stat_s[...])
    rank_s[...] = jnp.where(row16 == b, jnp.broadcast_to(rank, (B, P)),
                            rank_s[...])

    @pl.when(b == B - 1)
    def _finalize():
        rows = rank_s[...]
        bits = jax.lax.bitcast_convert_type(rows, jnp.int32)
        npos_col = stat_s[:, 2:3]
        k = jnp.minimum(NEGPOS * npos_col.astype(jnp.int32), P - 1)

        lo = jnp.zeros((B, 1), jnp.int32)
        hi = jnp.full((B, 1), jnp.int32(0x7F800000))
        for _ in range(31):
            mid = lo + (hi - lo + 1) // 2
            cnt = jnp.sum((bits >= mid).astype(jnp.int32), axis=1,
                          keepdims=True)
            take = cnt >= k
            lo = jnp.where(take, mid, lo)
            hi = jnp.where(take, hi, mid - 1)

        tval = jax.lax.bitcast_convert_type(lo, jnp.float32)
        gt = bits > lo
        cnt_gt = jnp.sum(gt.astype(jnp.int32), axis=1, keepdims=True)
        sum_gt = jnp.sum(jnp.where(gt, rows, 0.0), axis=1, keepdims=True)
        topk = sum_gt + (k - cnt_gt).astype(jnp.float32) * tval
        topk = jnp.where(k > 0, topk, 0.0)
        topk_total = jnp.sum(topk)

        S = jnp.sum(stat_s[...], axis=0, keepdims=True)
        lane1 = jax.lax.broadcasted_iota(jnp.int32, (1, 128), 1)

        def pick(i):
            return jnp.sum(jnp.where(lane1 == i, S, 0.0))

        N = jnp.maximum(pick(2), 1.0)
        N1 = jnp.maximum(pick(3), 1.0)
        out_ref[...] = jnp.concatenate([
            jnp.broadcast_to(pick(0) / N, (1, 128)),
            jnp.broadcast_to((pick(4) + topk_total) / N, (1, 128)),
            jnp.broadcast_to(pick(1) / N1, (1, 128)),
            jnp.zeros((5, 128), jnp.float32),
        ], axis=0)


def kernel(loc_data, conf_data, landm_data, priors, targets):
    xl = loc_data.transpose(0, 2, 1)
    xm = landm_data.transpose(0, 2, 1)
    xc = conf_data.transpose(0, 2, 1)

    pt = priors.T
    pxy, pwh = pt[0:2], pt[2:4]
    pf1 = pxy - pwh * 0.5
    pf2 = pxy + pwh * 0.5
    area_b = (pf2[0:1] - pf1[0:1]) * (pf2[1:2] - pf1[1:2])
    pri = jnp.concatenate([
        pxy,
        pwh,
        pf1, pf2,
        area_b,
        1.0 / (V0 * pwh),
        1.0 / pwh,
        jnp.zeros((3, P), jnp.float32),
    ], axis=0)

    tt = jnp.concatenate(
        [targets.transpose(0, 2, 1),
         jnp.zeros((B, 1, G), jnp.float32)], axis=1)

    bti, bto = pl.pallas_call(
        _match_body,
        grid=(B,),
        in_specs=[
            pl.BlockSpec((16, P), lambda b: (0, 0)),
            pl.BlockSpec((1, 16, G), lambda b: (b, 0, 0)),
        ],
        out_specs=[
            pl.BlockSpec((1, 1, P), lambda b: (b, 0, 0)),
            pl.BlockSpec((1, 1, P), lambda b: (b, 0, 0)),
        ],
        out_shape=[
            jax.ShapeDtypeStruct((B, 1, P), jnp.int32),
            jax.ShapeDtypeStruct((B, 1, P), jnp.float32),
        ],
    )(pri, tt)

    out = pl.pallas_call(
        _loss_body,
        grid=(B,),
        in_specs=[
            pl.BlockSpec((1, 4, P), lambda b: (b, 0, 0)),
            pl.BlockSpec((1, 10, P), lambda b: (b, 0, 0)),
            pl.BlockSpec((1, 2, P), lambda b: (b, 0, 0)),
            pl.BlockSpec((16, P), lambda b: (0, 0)),
            pl.BlockSpec((1, 16, G), lambda b: (b, 0, 0)),
            pl.BlockSpec((1, 1, P), lambda b: (b, 0, 0)),
            pl.BlockSpec((1, 1, P), lambda b: (b, 0, 0)),
        ],
        out_specs=pl.BlockSpec((8, 128), lambda b: (0, 0)),
        out_shape=jax.ShapeDtypeStruct((8, 128), jnp.float32),
        scratch_shapes=[
            pltpu.VMEM((B, P), jnp.float32),
            pltpu.VMEM((B, 128), jnp.float32),
        ],
    )(xl, xm, xc, pri, tt, bti, bto)

    return (out[0, 0], out[1, 0], out[2, 0])
